# baseline (speedup 1.0000x reference)
import functools

import jax
import jax.numpy as jnp
from jax import lax
from jax.experimental import pallas as pl
from jax.experimental.pallas import tpu as pltpu
from jax.experimental.pallas import tpu_sc as plsc

M, D, B = 1000000, 32, 16384
NC, NS, L = 2, 16, 16
CH = 128
EB = B // NS
OB = B // (NC * NS)
EK = EB // CH
OK = OB // CH

_mesh1 = plsc.VectorSubcoreMesh(
    core_axis_name="c", subcore_axis_name="s", num_cores=1, num_subcores=NS
)
_mesh2 = plsc.VectorSubcoreMesh(
    core_axis_name="c", subcore_axis_name="s", num_cores=NC, num_subcores=NS
)


@functools.partial(
    pl.kernel,
    out_type=jax.ShapeDtypeStruct((B // CH, CH), jnp.int32),
    mesh=_mesh1,
    compiler_params=pltpu.CompilerParams(use_tc_tiling_on_sc=False),
    scratch_types=[
        pltpu.VMEM((EK, CH), jnp.int32),
        pltpu.VMEM((EK, CH), jnp.int32),
        pltpu.VMEM((EK, CH), jnp.int32),
        pltpu.VMEM_SHARED((M,), jnp.int32),
    ],
)
def _sc_winners(idx2, w2, idx_b, pos_b, w_b, P):
    s = lax.axis_index("s")
    ebase = s * EB

    pltpu.sync_copy(idx2.at[pl.ds(s * EK, EK)], idx_b)

    for k in range(EK):
        for j in range(CH // L):
            sl = pl.ds(j * L, L)
            pos_b[k, sl] = ebase + k * CH + j * L + lax.iota(jnp.int32, L)

    @pl.loop(0, EK)
    def _(k):
        pltpu.sync_copy(pos_b.at[k], P.at[idx_b.at[k]])

    plsc.subcore_barrier()

    @pl.loop(0, EK)
    def _(k):
        pltpu.sync_copy(P.at[idx_b.at[k]], w_b.at[k])

    for k in range(EK):
        for j in range(CH // L):
            sl = pl.ds(j * L, L)
            w_b[k, sl] = w_b[k, sl] & (B - 1)

    pltpu.sync_copy(w_b, w2.at[pl.ds(s * EK, EK)])


@functools.partial(
    pl.kernel,
    out_type=jax.ShapeDtypeStruct((D, B), jnp.float32),
    mesh=_mesh2,
    compiler_params=pltpu.CompilerParams(use_tc_tiling_on_sc=False),
    scratch_types=[
        pltpu.VMEM((EK, CH), jnp.int32),
        pltpu.VMEM((EK, CH), jnp.int32),
        pltpu.VMEM((D, EB), jnp.float32),
        pltpu.VMEM((D, EB), jnp.float32),
        pltpu.VMEM((D, OB), jnp.float32),
        pltpu.VMEM_SHARED((D, B), jnp.float32),
    ],
)
def _sc_applyT(memT, idx2, w2, valT, outT,
               idx_b, w_b, vT_b, gT_b, sT_b, accT):
    c = lax.axis_index("c")
    s = lax.axis_index("s")
    ebase = s * EB

    pltpu.sync_copy(idx2.at[pl.ds(s * EK, EK)], idx_b)
    pltpu.sync_copy(w2.at[pl.ds(s * EK, EK)], w_b)

    @pl.loop(0, D)
    def _(d):
        pltpu.sync_copy(valT.at[d, pl.ds(ebase, EB)], vT_b.at[d])

    @pl.loop(0, D)
    def _(d):
        @pl.loop(0, EK)
        def _(k):
            pltpu.sync_copy(memT.at[d].at[idx_b.at[k]],
                            gT_b.at[d, pl.ds(k * CH, CH)])

    @pl.loop(0, D)
    def _(d):
        pltpu.sync_copy(gT_b.at[d], accT.at[d, pl.ds(ebase, EB)])

    plsc.subcore_barrier()

    @pl.loop(0, D)
    def _(d):
        @pl.loop(0, EK)
        def _(k):
            pltpu.sync_copy(vT_b.at[d, pl.ds(k * CH, CH)],
                            accT.at[d].at[w_b.at[k]], add=True)

    plsc.subcore_barrier()

    obase = ebase + c * OB
    row0 = c * OK

    @pl.loop(0, D)
    def _(d):
        @pl.loop(0, OK)
        def _(k):
            pltpu.sync_copy(accT.at[d].at[w_b.at[row0 + k]],
                            sT_b.at[d, pl.ds(k * CH, CH)])

    @pl.loop(0, D)
    def _(d):
        pltpu.sync_copy(sT_b.at[d], outT.at[d, pl.ds(obase, OB)])


def kernel(mem, idx, val):
    idx2 = idx.reshape(B // CH, CH)
    w2 = _sc_winners(idx2)
    outT = _sc_applyT(mem.T, idx2, w2, val.T)
    return outT.T


# speedup vs baseline: 5.2353x; 5.2353x over previous
import functools

import jax
import jax.numpy as jnp
from jax import lax
from jax.experimental import pallas as pl
from jax.experimental.pallas import tpu as pltpu
from jax.experimental.pallas import tpu_sc as plsc

M, D, B = 1000000, 32, 16384
NC, NS, L = 2, 16, 16
CH = 128
EB = B // NS
OB = B // (NC * NS)
EK = EB // CH
OK = OB // CH

_mesh1 = plsc.VectorSubcoreMesh(
    core_axis_name="c", subcore_axis_name="s", num_cores=1, num_subcores=NS
)
_mesh2 = plsc.VectorSubcoreMesh(
    core_axis_name="c", subcore_axis_name="s", num_cores=NC, num_subcores=NS
)


@functools.partial(
    pl.kernel,
    out_type=jax.ShapeDtypeStruct((B // CH, CH), jnp.int32),
    mesh=_mesh1,
    compiler_params=pltpu.CompilerParams(use_tc_tiling_on_sc=False),
    scratch_types=[
        pltpu.VMEM((EK, CH), jnp.int32),
        pltpu.VMEM((EK, CH), jnp.int32),
        pltpu.VMEM((EK, CH), jnp.int32),
        pltpu.VMEM_SHARED((M,), jnp.int32),
    ],
)
def _sc_winners(idx2, w2, idx_b, pos_b, w_b, P):
    s = lax.axis_index("s")
    ebase = s * EB

    pltpu.sync_copy(idx2.at[pl.ds(s * EK, EK)], idx_b)

    for k in range(EK):
        for j in range(CH // L):
            sl = pl.ds(j * L, L)
            pos_b[k, sl] = ebase + k * CH + j * L + lax.iota(jnp.int32, L)

    @pl.loop(0, EK)
    def _(k):
        pltpu.sync_copy(pos_b.at[k], P.at[idx_b.at[k]])

    plsc.subcore_barrier()

    @pl.loop(0, EK)
    def _(k):
        pltpu.sync_copy(P.at[idx_b.at[k]], w_b.at[k])

    for k in range(EK):
        for j in range(CH // L):
            sl = pl.ds(j * L, L)
            w_b[k, sl] = w_b[k, sl] & (B - 1)

    pltpu.sync_copy(w_b, w2.at[pl.ds(s * EK, EK)])


@functools.partial(
    pl.kernel,
    out_type=jax.ShapeDtypeStruct((B, D), jnp.float32),
    mesh=_mesh2,
    compiler_params=pltpu.CompilerParams(use_tc_tiling_on_sc=False),
    scratch_types=[
        pltpu.VMEM((EK, CH), jnp.int32),
        pltpu.VMEM((EK, CH), jnp.int32),
        pltpu.VMEM((EB, D), jnp.float32),
        pltpu.VMEM((CH, D), jnp.float32),
        pltpu.VMEM((OB, D), jnp.float32),
        pltpu.VMEM((OB, D), jnp.float32),
        pltpu.VMEM_SHARED((B, D), jnp.float32),
    ],
)
def _sc_apply(mem, idx2, w2, val, out,
              idx_b, w_b, val_b, zbuf, g_b, s_b, acc):
    c = lax.axis_index("c")
    s = lax.axis_index("s")
    ebase = s * EB

    pltpu.sync_copy(idx2.at[pl.ds(s * EK, EK)], idx_b)
    pltpu.sync_copy(w2.at[pl.ds(s * EK, EK)], w_b)
    pltpu.sync_copy(val.at[pl.ds(ebase, EB)], val_b)

    zrow = jnp.zeros((L,), jnp.float32)
    for r in range(CH):
        for j in range(D // L):
            zbuf[r, pl.ds(j * L, L)] = zrow

    @pl.loop(0, EK)
    def _(k):
        pltpu.sync_copy(zbuf, acc.at[pl.ds(ebase + k * CH, CH)])

    plsc.subcore_barrier()

    @pl.loop(0, EK)
    def _(k):
        pltpu.sync_copy(val_b.at[pl.ds(k * CH, CH)], acc.at[w_b.at[k]],
                        add=True)

    obase = ebase + c * OB
    row0 = c * OK

    @pl.loop(0, OK)
    def _(k):
        pltpu.sync_copy(mem.at[idx_b.at[row0 + k]], g_b.at[pl.ds(k * CH, CH)])

    plsc.subcore_barrier()

    @pl.loop(0, OK)
    def _(k):
        pltpu.sync_copy(acc.at[w_b.at[row0 + k]], s_b.at[pl.ds(k * CH, CH)])

    @pl.loop(0, OB)
    def _(r):
        for j in range(D // L):
            sl = pl.ds(j * L, L)
            g_b[r, sl] = g_b[r, sl] + s_b[r, sl]

    pltpu.sync_copy(g_b, out.at[pl.ds(obase, OB)])


def kernel(mem, idx, val):
    idx2 = idx.reshape(B // CH, CH)
    w2 = _sc_winners(idx2)
    return _sc_apply(mem, idx2, w2, val)
